# Initial kernel scaffold; baseline (speedup 1.0000x reference)
import jax
import jax.numpy as jnp
from jax.experimental import pallas as pl


def kernel(pos, edge_index, batch, p_local1, p_local2, p_global, p_head):
    raise NotImplementedError("write your pallas kernel here")


# speedup vs baseline: 1.0347x; 1.0347x over previous
import functools

import jax
import jax.numpy as jnp
from jax import lax
from jax.experimental import pallas as pl
from jax.experimental.pallas import tpu as pltpu

_EPS = 1e-5


def _ln(x, g, b):
    mu = jnp.mean(x, axis=-1, keepdims=True)
    var = jnp.mean((x - mu) ** 2, axis=-1, keepdims=True)
    return (x - mu) * lax.rsqrt(var + _EPS) * g + b


def _relu(x):
    return jnp.maximum(x, 0.0)


def _mlp1_body(msg_ref, g1, b1, W1, g2, b2, W2, g3, b3, W3, out_ref):
    m = msg_ref[...]
    mask3 = (lax.broadcasted_iota(jnp.int32, m.shape, 1) < 3).astype(jnp.float32)
    mu = jnp.sum(m, axis=1, keepdims=True) * (1.0 / 3.0)
    var = jnp.sum(((m - mu) ** 2) * mask3, axis=1, keepdims=True) * (1.0 / 3.0)
    a = _relu((m - mu) * lax.rsqrt(var + _EPS) * g1[...] + b1[...])
    h = jnp.dot(a, W1[...], preferred_element_type=jnp.float32)
    h = jnp.dot(_relu(_ln(h, g2[...], b2[...])), W2[...], preferred_element_type=jnp.float32)
    h = jnp.dot(_relu(_ln(h, g3[...], b3[...])), W3[...], preferred_element_type=jnp.float32)
    out_ref[...] = h


def _run_mlp1(msgp, p1, Eb):
    Ep = msgp.shape[0]
    (g1, b1, W1), (g2, b2, W2), (g3, b3, W3) = p1
    g1p = jnp.concatenate([g1, jnp.ones((5,), jnp.float32)]).reshape(1, 8)
    b1p = jnp.concatenate([b1, jnp.zeros((5,), jnp.float32)]).reshape(1, 8)
    W1p = jnp.concatenate([W1, jnp.zeros((5, W1.shape[1]), jnp.float32)], axis=0)
    d1, d2, d3 = W1.shape[1], W2.shape[1], W3.shape[1]
    grid = (Ep // Eb,)
    full = lambda shape: pl.BlockSpec(shape, lambda i: (0, 0))
    return pl.pallas_call(
        _mlp1_body,
        grid=grid,
        in_specs=[
            pl.BlockSpec((Eb, 8), lambda i: (i, 0)),
            full((1, 8)), full((1, 8)), full((8, d1)),
            full((1, d1)), full((1, d1)), full((d1, d2)),
            full((1, d2)), full((1, d2)), full((d2, d3)),
        ],
        out_specs=pl.BlockSpec((Eb, d3), lambda i: (i, 0)),
        out_shape=jax.ShapeDtypeStruct((Ep, d3), jnp.float32),
    )(msgp, g1p, b1p, W1p, g2.reshape(1, -1), b2.reshape(1, -1), W2,
      g3.reshape(1, -1), b3.reshape(1, -1), W3)


def _mlp2_body(xj_ref, msg_ref, g1x, b1x, g1m, b1m, W1a, W1b,
               g2, b2, W2, g3, b3, W3, out_ref):
    xj = xj_ref[...]
    m = msg_ref[...]
    C = xj.shape[1] + 3.0
    mask3 = (lax.broadcasted_iota(jnp.int32, m.shape, 1) < 3).astype(jnp.float32)
    s = jnp.sum(xj, axis=1, keepdims=True) + jnp.sum(m, axis=1, keepdims=True)
    mu = s / C
    var = (jnp.sum((xj - mu) ** 2, axis=1, keepdims=True)
           + jnp.sum(((m - mu) ** 2) * mask3, axis=1, keepdims=True)) / C
    inv = lax.rsqrt(var + _EPS)
    ax = _relu((xj - mu) * inv * g1x[...] + b1x[...])
    am = _relu((m - mu) * inv * g1m[...] + b1m[...])
    h = (jnp.dot(ax, W1a[...], preferred_element_type=jnp.float32)
         + jnp.dot(am, W1b[...], preferred_element_type=jnp.float32))
    h = jnp.dot(_relu(_ln(h, g2[...], b2[...])), W2[...], preferred_element_type=jnp.float32)
    h = jnp.dot(_relu(_ln(h, g3[...], b3[...])), W3[...], preferred_element_type=jnp.float32)
    out_ref[...] = h


def _run_mlp2(xjp, msgp, p2, Eb):
    Ep, F = xjp.shape
    (g1, b1, W1), (g2, b2, W2), (g3, b3, W3) = p2
    g1x, g1m = g1[:F].reshape(1, F), jnp.concatenate(
        [g1[F:], jnp.ones((5,), jnp.float32)]).reshape(1, 8)
    b1x, b1m = b1[:F].reshape(1, F), jnp.concatenate(
        [b1[F:], jnp.zeros((5,), jnp.float32)]).reshape(1, 8)
    W1a = W1[:F]
    W1b = jnp.concatenate([W1[F:], jnp.zeros((5, W1.shape[1]), jnp.float32)], axis=0)
    d1, d2, d3 = W1.shape[1], W2.shape[1], W3.shape[1]
    grid = (Ep // Eb,)
    full = lambda shape: pl.BlockSpec(shape, lambda i: (0, 0))
    return pl.pallas_call(
        _mlp2_body,
        grid=grid,
        in_specs=[
            pl.BlockSpec((Eb, F), lambda i: (i, 0)),
            pl.BlockSpec((Eb, 8), lambda i: (i, 0)),
            full((1, F)), full((1, F)), full((1, 8)), full((1, 8)),
            full((F, d1)), full((8, d1)),
            full((1, d1)), full((1, d1)), full((d1, d2)),
            full((1, d2)), full((1, d2)), full((d2, d3)),
        ],
        out_specs=pl.BlockSpec((Eb, d3), lambda i: (i, 0)),
        out_shape=jax.ShapeDtypeStruct((Ep, d3), jnp.float32),
    )(xjp, msgp, g1x, b1x, g1m, b1m, W1a, W1b,
      g2.reshape(1, -1), b2.reshape(1, -1), W2,
      g3.reshape(1, -1), b3.reshape(1, -1), W3)


def _mlp3_body(x2_ref, pos_ref, batch_ref,
               g1x, b1x, g1p, b1p, W1a, W1b, g2, b2, W2, g3, b3, W3,
               hg1, hb1, HW1, hc1, hg2, hb2, HW2, hc2, hg3, hb3, HW3, hc3,
               out_ref, pooled_ref):
    i = pl.program_id(0)
    nblk = pl.num_programs(0)

    @pl.when(i == 0)
    def _():
        pooled_ref[...] = jnp.full_like(pooled_ref, -jnp.inf)

    x2 = x2_ref[...]
    p = pos_ref[...]
    C = x2.shape[1] + 3.0
    mask3 = (lax.broadcasted_iota(jnp.int32, p.shape, 1) < 3).astype(jnp.float32)
    mu = (jnp.sum(x2, axis=1, keepdims=True) + jnp.sum(p, axis=1, keepdims=True)) / C
    var = (jnp.sum((x2 - mu) ** 2, axis=1, keepdims=True)
           + jnp.sum(((p - mu) ** 2) * mask3, axis=1, keepdims=True)) / C
    inv = lax.rsqrt(var + _EPS)
    ax = _relu((x2 - mu) * inv * g1x[...] + b1x[...])
    ap = _relu((p - mu) * inv * g1p[...] + b1p[...])
    h = (jnp.dot(ax, W1a[...], preferred_element_type=jnp.float32)
         + jnp.dot(ap, W1b[...], preferred_element_type=jnp.float32))
    h = jnp.dot(_relu(_ln(h, g2[...], b2[...])), W2[...], preferred_element_type=jnp.float32)
    g_out = jnp.dot(_relu(_ln(h, g3[...], b3[...])), W3[...], preferred_element_type=jnp.float32)

    b0 = batch_ref[...][:, :1]
    for bb in range(8):
        mask = b0 == bb
        contrib = jnp.max(jnp.where(mask, g_out, -jnp.inf), axis=0, keepdims=True)
        pooled_ref[bb:bb + 1, :] = jnp.maximum(pooled_ref[bb:bb + 1, :], contrib)

    @pl.when(i == nblk - 1)
    def _():
        pool = pooled_ref[...]
        o = jnp.dot(_relu(_ln(pool, hg1[...], hb1[...])), HW1[...],
                    preferred_element_type=jnp.float32) + hc1[...]
        o = jnp.dot(_relu(_ln(o, hg2[...], hb2[...])), HW2[...],
                    preferred_element_type=jnp.float32) + hc2[...]
        o = jnp.dot(_relu(_ln(o, hg3[...], hb3[...])), HW3[...],
                    preferred_element_type=jnp.float32) + hc3[...]
        out_ref[...] = o


def _run_mlp3(x2p, posp, batchp, pg, ph, Nb):
    Np, F2 = x2p.shape
    (g1, b1, W1), (g2, b2, W2), (g3, b3, W3) = pg
    (hg1, hb1, HW1, hc1), (hg2, hb2, HW2, hc2), (hg3, hb3, HW3, hc3) = ph
    g1x, g1p = g1[:F2].reshape(1, F2), jnp.concatenate(
        [g1[F2:], jnp.ones((5,), jnp.float32)]).reshape(1, 8)
    b1x, b1p = b1[:F2].reshape(1, F2), jnp.concatenate(
        [b1[F2:], jnp.zeros((5,), jnp.float32)]).reshape(1, 8)
    W1a = W1[:F2]
    W1b = jnp.concatenate([W1[F2:], jnp.zeros((5, W1.shape[1]), jnp.float32)], axis=0)
    d1, d2, d3 = W1.shape[1], W2.shape[1], W3.shape[1]
    e1, e2, e3 = HW1.shape[1], HW2.shape[1], HW3.shape[1]
    grid = (Np // Nb,)
    full = lambda shape: pl.BlockSpec(shape, lambda i: (0, 0))
    r2 = lambda v: v.reshape(1, -1)
    return pl.pallas_call(
        _mlp3_body,
        grid=grid,
        in_specs=[
            pl.BlockSpec((Nb, F2), lambda i: (i, 0)),
            pl.BlockSpec((Nb, 8), lambda i: (i, 0)),
            pl.BlockSpec((Nb, 8), lambda i: (i, 0)),
            full((1, F2)), full((1, F2)), full((1, 8)), full((1, 8)),
            full((F2, d1)), full((8, d1)),
            full((1, d1)), full((1, d1)), full((d1, d2)),
            full((1, d2)), full((1, d2)), full((d2, d3)),
            full((1, d3)), full((1, d3)), full((d3, e1)), full((1, e1)),
            full((1, e1)), full((1, e1)), full((e1, e2)), full((1, e2)),
            full((1, e2)), full((1, e2)), full((e2, e3)), full((1, e3)),
        ],
        out_specs=pl.BlockSpec((8, e3), lambda i: (0, 0)),
        out_shape=jax.ShapeDtypeStruct((8, e3), jnp.float32),
        scratch_shapes=[pltpu.VMEM((8, d3), jnp.float32)],
    )(x2p, posp, batchp, g1x, b1x, g1p, b1p, W1a, W1b,
      r2(g2), r2(b2), W2, r2(g3), r2(b3), W3,
      r2(hg1), r2(hb1), HW1, r2(hc1), r2(hg2), r2(hb2), HW2, r2(hc2),
      r2(hg3), r2(hb3), HW3, r2(hc3))


def _ceil_to(x, m):
    return (x + m - 1) // m * m


def kernel(pos, edge_index, batch, p_local1, p_local2, p_global, p_head):
    N = pos.shape[0]
    E = edge_index.shape[1]
    EE = E + N
    Eb = 2048
    Nb = 1024 if N >= 1024 else 256
    Ep = _ceil_to(EE, Eb)
    Np = _ceil_to(N, Nb)

    loops = jnp.arange(N, dtype=edge_index.dtype)
    row = jnp.concatenate([edge_index[0], loops])
    col = jnp.concatenate([edge_index[1], loops])

    deg = jax.ops.segment_sum(jnp.ones(EE, jnp.float32), row, num_segments=N)
    ew = (1.0 / deg)[row]
    rel = pos[row] - pos[col]
    msg = ew[:, None] * rel
    msgp = jnp.zeros((Ep, 8), jnp.float32).at[:EE, :3].set(msg)

    h = _run_mlp1(msgp, p_local1, Eb)
    x = jax.ops.segment_max(h[:EE], col, num_segments=N)

    xjp = jnp.zeros((Ep, x.shape[1]), jnp.float32).at[:EE].set(x[row])
    h2 = _run_mlp2(xjp, msgp, p_local2, Eb)
    x2 = jax.ops.segment_max(h2[:EE], col, num_segments=N)

    x2p = jnp.full((Np, x2.shape[1]), 0.0, jnp.float32).at[:N].set(x2)
    posp = jnp.zeros((Np, 8), jnp.float32).at[:N, :3].set(pos)
    batchp = jnp.full((Np, 8), 8, jnp.int32).at[:N, :].set(batch[:, None])

    out = _run_mlp3(x2p, posp, batchp, p_global, p_head, Nb)
    return out


# speedup vs baseline: 1.6370x; 1.5822x over previous
import functools

import jax
import jax.numpy as jnp
from jax import lax
from jax.experimental import pallas as pl
from jax.experimental.pallas import tpu as pltpu
from jax.experimental.pallas import tpu_sc as plsc

_EPS = 1e-5
_NC, _NS, _L = 2, 16, 16
_NW = _NC * _NS


def _sc_mesh():
    return plsc.VectorSubcoreMesh(core_axis_name="c", subcore_axis_name="s")


_SC_PARAMS = pltpu.CompilerParams(needs_layout_passes=False)


def _wid():
    return lax.axis_index("s") * _NC + lax.axis_index("c")


def _iota16():
    return lax.iota(jnp.int32, _L)


def _sc_deg(rowd, Np):
    Ed = rowd.shape[0]
    S = 4096
    Cn = Np // _NW

    @functools.partial(
        pl.kernel,
        out_type=jax.ShapeDtypeStruct((Np,), jnp.float32),
        mesh=_sc_mesh(),
        compiler_params=_SC_PARAMS,
        scratch_types=[pltpu.VMEM((Cn,), jnp.float32),
                       pltpu.VMEM((S,), jnp.int32)],
    )
    def k(row_hbm, deg_hbm, ldeg, rbuf):
        wid = _wid()
        lo = wid * Cn
        ones = jnp.ones((_L,), jnp.float32)

        def init(i, _):
            ldeg[pl.ds(i * _L, _L)] = ones
            return 0
        lax.fori_loop(0, Cn // _L, init, 0)

        def chunk(ci, _):
            pltpu.sync_copy(row_hbm.at[pl.ds(ci * S, S)], rbuf)

            def grp(g, _):
                r = rbuf[pl.ds(g * _L, _L)]
                msk = (r >= lo) & (r < lo + Cn)
                plsc.addupdate_scatter(ldeg, [r - lo], ones, mask=msk)
                return 0
            lax.fori_loop(0, S // _L, grp, 0)
            return 0
        lax.fori_loop(0, Ed // S, chunk, 0)
        pltpu.sync_copy(ldeg, deg_hbm.at[pl.ds(lo, Cn)])

    return k(rowd)


def _sc_msg(rowp, colp, px, py, pz, deg):
    Ep = rowp.shape[0]
    Np = px.shape[0]
    Te = Ep // _NW
    S = 672 if Te % 672 == 0 else 16
    nch = Te // S

    @functools.partial(
        pl.kernel,
        out_type=jax.ShapeDtypeStruct((Ep * 8,), jnp.float32),
        mesh=_sc_mesh(),
        compiler_params=_SC_PARAMS,
        scratch_types=[pltpu.VMEM((Np,), jnp.float32),
                       pltpu.VMEM((Np,), jnp.float32),
                       pltpu.VMEM((Np,), jnp.float32),
                       pltpu.VMEM((Np,), jnp.float32),
                       pltpu.VMEM((S,), jnp.int32),
                       pltpu.VMEM((S,), jnp.int32),
                       pltpu.VMEM((S * 8,), jnp.float32)],
    )
    def k(row_hbm, col_hbm, px_hbm, py_hbm, pz_hbm, deg_hbm, msg_hbm,
          pxv, pyv, pzv, dgv, rbuf, cbuf, obuf):
        wid = _wid()
        base0 = wid * Te
        pltpu.sync_copy(px_hbm, pxv)
        pltpu.sync_copy(py_hbm, pyv)
        pltpu.sync_copy(pz_hbm, pzv)
        pltpu.sync_copy(deg_hbm, dgv)
        zz = jnp.zeros((_L,), jnp.float32)

        def zinit(i, _):
            obuf[pl.ds(i * _L, _L)] = zz
            return 0
        lax.fori_loop(0, S * 8 // _L, zinit, 0)

        def chunk(ci, _):
            b = base0 + ci * S
            pltpu.sync_copy(row_hbm.at[pl.ds(b, S)], rbuf)
            pltpu.sync_copy(col_hbm.at[pl.ds(b, S)], cbuf)

            def grp(g, _):
                r = rbuf[pl.ds(g * _L, _L)]
                c = cbuf[pl.ds(g * _L, _L)]
                ew = 1.0 / plsc.load_gather(dgv, [r])
                dx = plsc.load_gather(pxv, [r]) - plsc.load_gather(pxv, [c])
                dy = plsc.load_gather(pyv, [r]) - plsc.load_gather(pyv, [c])
                dz = plsc.load_gather(pzv, [r]) - plsc.load_gather(pzv, [c])
                bi = (g * _L + _iota16()) * 8
                plsc.store_scatter(obuf, [bi], ew * dx)
                plsc.store_scatter(obuf, [bi + 1], ew * dy)
                plsc.store_scatter(obuf, [bi + 2], ew * dz)
                return 0
            lax.fori_loop(0, S // _L, grp, 0)
            pltpu.sync_copy(obuf, msg_hbm.at[pl.ds(b * 8, S * 8)])
            return 0
        lax.fori_loop(0, nch, chunk, 0)

    return k(rowp, colp, px, py, pz, deg)


def _sc_gather_rows(x, rowp):
    Np, F = x.shape
    Ep = rowp.shape[0]
    Te = Ep // _NW
    K = 128
    nch = Te // K

    @functools.partial(
        pl.kernel,
        out_type=jax.ShapeDtypeStruct((Ep, F), jnp.float32),
        mesh=_sc_mesh(),
        compiler_params=_SC_PARAMS,
        scratch_types=[pltpu.VMEM((K,), jnp.int32),
                       pltpu.VMEM((K, F), jnp.float32),
                       pltpu.SemaphoreType.DMA],
    )
    def k(x_hbm, row_hbm, out_hbm, idxv, gbuf, sem):
        wid = _wid()
        base0 = wid * Te

        def chunk(ci, _):
            b = base0 + ci * K
            pltpu.sync_copy(row_hbm.at[pl.ds(b, K)], idxv)
            pltpu.async_copy(x_hbm.at[idxv], gbuf, sem).wait()
            pltpu.sync_copy(gbuf, out_hbm.at[pl.ds(b, K)])
            return 0
        lax.fori_loop(0, nch, chunk, 0)

    return k(x, rowp)


def _ln(x, g, b):
    mu = jnp.mean(x, axis=-1, keepdims=True)
    var = jnp.mean((x - mu) ** 2, axis=-1, keepdims=True)
    return (x - mu) * lax.rsqrt(var + _EPS) * g + b


def _relu(x):
    return jnp.maximum(x, 0.0)


def _mlp1_body(msg_ref, g1, b1, W1, g2, b2, W2, g3, b3, W3, out_ref):
    m = msg_ref[...]
    mask3 = (lax.broadcasted_iota(jnp.int32, m.shape, 1) < 3).astype(jnp.float32)
    mu = jnp.sum(m, axis=1, keepdims=True) * (1.0 / 3.0)
    var = jnp.sum(((m - mu) ** 2) * mask3, axis=1, keepdims=True) * (1.0 / 3.0)
    a = _relu((m - mu) * lax.rsqrt(var + _EPS) * g1[...] + b1[...])
    h = jnp.dot(a, W1[...], preferred_element_type=jnp.float32)
    h = jnp.dot(_relu(_ln(h, g2[...], b2[...])), W2[...], preferred_element_type=jnp.float32)
    h = jnp.dot(_relu(_ln(h, g3[...], b3[...])), W3[...], preferred_element_type=jnp.float32)
    out_ref[...] = h


def _run_mlp1(msgp, p1, Eb):
    Ep = msgp.shape[0]
    (g1, b1, W1), (g2, b2, W2), (g3, b3, W3) = p1
    g1p = jnp.concatenate([g1, jnp.ones((5,), jnp.float32)]).reshape(1, 8)
    b1p = jnp.concatenate([b1, jnp.zeros((5,), jnp.float32)]).reshape(1, 8)
    W1p = jnp.concatenate([W1, jnp.zeros((5, W1.shape[1]), jnp.float32)], axis=0)
    d1, d2, d3 = W1.shape[1], W2.shape[1], W3.shape[1]
    grid = (Ep // Eb,)
    full = lambda shape: pl.BlockSpec(shape, lambda i: (0, 0))
    return pl.pallas_call(
        _mlp1_body,
        grid=grid,
        in_specs=[
            pl.BlockSpec((Eb, 8), lambda i: (i, 0)),
            full((1, 8)), full((1, 8)), full((8, d1)),
            full((1, d1)), full((1, d1)), full((d1, d2)),
            full((1, d2)), full((1, d2)), full((d2, d3)),
        ],
        out_specs=pl.BlockSpec((Eb, d3), lambda i: (i, 0)),
        out_shape=jax.ShapeDtypeStruct((Ep, d3), jnp.float32),
    )(msgp, g1p, b1p, W1p, g2.reshape(1, -1), b2.reshape(1, -1), W2,
      g3.reshape(1, -1), b3.reshape(1, -1), W3)


def _mlp2_body(xj_ref, msg_ref, g1x, b1x, g1m, b1m, W1a, W1b,
               g2, b2, W2, g3, b3, W3, out_ref):
    xj = xj_ref[...]
    m = msg_ref[...]
    C = xj.shape[1] + 3.0
    mask3 = (lax.broadcasted_iota(jnp.int32, m.shape, 1) < 3).astype(jnp.float32)
    s = jnp.sum(xj, axis=1, keepdims=True) + jnp.sum(m, axis=1, keepdims=True)
    mu = s / C
    var = (jnp.sum((xj - mu) ** 2, axis=1, keepdims=True)
           + jnp.sum(((m - mu) ** 2) * mask3, axis=1, keepdims=True)) / C
    inv = lax.rsqrt(var + _EPS)
    ax = _relu((xj - mu) * inv * g1x[...] + b1x[...])
    am = _relu((m - mu) * inv * g1m[...] + b1m[...])
    h = (jnp.dot(ax, W1a[...], preferred_element_type=jnp.float32)
         + jnp.dot(am, W1b[...], preferred_element_type=jnp.float32))
    h = jnp.dot(_relu(_ln(h, g2[...], b2[...])), W2[...], preferred_element_type=jnp.float32)
    h = jnp.dot(_relu(_ln(h, g3[...], b3[...])), W3[...], preferred_element_type=jnp.float32)
    out_ref[...] = h


def _run_mlp2(xjp, msgp, p2, Eb):
    Ep, F = xjp.shape
    (g1, b1, W1), (g2, b2, W2), (g3, b3, W3) = p2
    g1x, g1m = g1[:F].reshape(1, F), jnp.concatenate(
        [g1[F:], jnp.ones((5,), jnp.float32)]).reshape(1, 8)
    b1x, b1m = b1[:F].reshape(1, F), jnp.concatenate(
        [b1[F:], jnp.zeros((5,), jnp.float32)]).reshape(1, 8)
    W1a = W1[:F]
    W1b = jnp.concatenate([W1[F:], jnp.zeros((5, W1.shape[1]), jnp.float32)], axis=0)
    d1, d2, d3 = W1.shape[1], W2.shape[1], W3.shape[1]
    grid = (Ep // Eb,)
    full = lambda shape: pl.BlockSpec(shape, lambda i: (0, 0))
    return pl.pallas_call(
        _mlp2_body,
        grid=grid,
        in_specs=[
            pl.BlockSpec((Eb, F), lambda i: (i, 0)),
            pl.BlockSpec((Eb, 8), lambda i: (i, 0)),
            full((1, F)), full((1, F)), full((1, 8)), full((1, 8)),
            full((F, d1)), full((8, d1)),
            full((1, d1)), full((1, d1)), full((d1, d2)),
            full((1, d2)), full((1, d2)), full((d2, d3)),
        ],
        out_specs=pl.BlockSpec((Eb, d3), lambda i: (i, 0)),
        out_shape=jax.ShapeDtypeStruct((Ep, d3), jnp.float32),
    )(xjp, msgp, g1x, b1x, g1m, b1m, W1a, W1b,
      g2.reshape(1, -1), b2.reshape(1, -1), W2,
      g3.reshape(1, -1), b3.reshape(1, -1), W3)


def _mlp3_body(x2_ref, pos_ref, batch_ref,
               g1x, b1x, g1p, b1p, W1a, W1b, g2, b2, W2, g3, b3, W3,
               hg1, hb1, HW1, hc1, hg2, hb2, HW2, hc2, hg3, hb3, HW3, hc3,
               out_ref, pooled_ref):
    i = pl.program_id(0)
    nblk = pl.num_programs(0)

    @pl.when(i == 0)
    def _():
        pooled_ref[...] = jnp.full_like(pooled_ref, -jnp.inf)

    x2 = x2_ref[...]
    p = pos_ref[...]
    C = x2.shape[1] + 3.0
    mask3 = (lax.broadcasted_iota(jnp.int32, p.shape, 1) < 3).astype(jnp.float32)
    mu = (jnp.sum(x2, axis=1, keepdims=True) + jnp.sum(p, axis=1, keepdims=True)) / C
    var = (jnp.sum((x2 - mu) ** 2, axis=1, keepdims=True)
           + jnp.sum(((p - mu) ** 2) * mask3, axis=1, keepdims=True)) / C
    inv = lax.rsqrt(var + _EPS)
    ax = _relu((x2 - mu) * inv * g1x[...] + b1x[...])
    ap = _relu((p - mu) * inv * g1p[...] + b1p[...])
    h = (jnp.dot(ax, W1a[...], preferred_element_type=jnp.float32)
         + jnp.dot(ap, W1b[...], preferred_element_type=jnp.float32))
    h = jnp.dot(_relu(_ln(h, g2[...], b2[...])), W2[...], preferred_element_type=jnp.float32)
    g_out = jnp.dot(_relu(_ln(h, g3[...], b3[...])), W3[...], preferred_element_type=jnp.float32)

    b0 = batch_ref[...][:, :1]
    for bb in range(8):
        mask = b0 == bb
        contrib = jnp.max(jnp.where(mask, g_out, -jnp.inf), axis=0, keepdims=True)
        pooled_ref[bb:bb + 1, :] = jnp.maximum(pooled_ref[bb:bb + 1, :], contrib)

    @pl.when(i == nblk - 1)
    def _():
        pool = pooled_ref[...]
        o = jnp.dot(_relu(_ln(pool, hg1[...], hb1[...])), HW1[...],
                    preferred_element_type=jnp.float32) + hc1[...]
        o = jnp.dot(_relu(_ln(o, hg2[...], hb2[...])), HW2[...],
                    preferred_element_type=jnp.float32) + hc2[...]
        o = jnp.dot(_relu(_ln(o, hg3[...], hb3[...])), HW3[...],
                    preferred_element_type=jnp.float32) + hc3[...]
        out_ref[...] = o


def _run_mlp3(x2p, posp, batchp, pg, ph, Nb):
    Np, F2 = x2p.shape
    (g1, b1, W1), (g2, b2, W2), (g3, b3, W3) = pg
    (hg1, hb1, HW1, hc1), (hg2, hb2, HW2, hc2), (hg3, hb3, HW3, hc3) = ph
    g1x, g1p = g1[:F2].reshape(1, F2), jnp.concatenate(
        [g1[F2:], jnp.ones((5,), jnp.float32)]).reshape(1, 8)
    b1x, b1p = b1[:F2].reshape(1, F2), jnp.concatenate(
        [b1[F2:], jnp.zeros((5,), jnp.float32)]).reshape(1, 8)
    W1a = W1[:F2]
    W1b = jnp.concatenate([W1[F2:], jnp.zeros((5, W1.shape[1]), jnp.float32)], axis=0)
    d1, d2, d3 = W1.shape[1], W2.shape[1], W3.shape[1]
    e1, e2, e3 = HW1.shape[1], HW2.shape[1], HW3.shape[1]
    grid = (Np // Nb,)
    full = lambda shape: pl.BlockSpec(shape, lambda i: (0, 0))
    r2 = lambda v: v.reshape(1, -1)
    return pl.pallas_call(
        _mlp3_body,
        grid=grid,
        in_specs=[
            pl.BlockSpec((Nb, F2), lambda i: (i, 0)),
            pl.BlockSpec((Nb, 8), lambda i: (i, 0)),
            pl.BlockSpec((Nb, 8), lambda i: (i, 0)),
            full((1, F2)), full((1, F2)), full((1, 8)), full((1, 8)),
            full((F2, d1)), full((8, d1)),
            full((1, d1)), full((1, d1)), full((d1, d2)),
            full((1, d2)), full((1, d2)), full((d2, d3)),
            full((1, d3)), full((1, d3)), full((d3, e1)), full((1, e1)),
            full((1, e1)), full((1, e1)), full((e1, e2)), full((1, e2)),
            full((1, e2)), full((1, e2)), full((e2, e3)), full((1, e3)),
        ],
        out_specs=pl.BlockSpec((8, e3), lambda i: (0, 0)),
        out_shape=jax.ShapeDtypeStruct((8, e3), jnp.float32),
        scratch_shapes=[pltpu.VMEM((8, d3), jnp.float32)],
    )(x2p, posp, batchp, g1x, b1x, g1p, b1p, W1a, W1b,
      r2(g2), r2(b2), W2, r2(g3), r2(b3), W3,
      r2(hg1), r2(hb1), HW1, r2(hc1), r2(hg2), r2(hb2), HW2, r2(hc2),
      r2(hg3), r2(hb3), HW3, r2(hc3))


def _ceil_to(x, m):
    return (x + m - 1) // m * m


def kernel(pos, edge_index, batch, p_local1, p_local2, p_global, p_head):
    N = pos.shape[0]
    E = edge_index.shape[1]
    EE = E + N
    Eb = 2048
    Nb = 1024 if N >= 1024 else 256
    Ep = _ceil_to(EE, Eb)
    Np = _ceil_to(N, Nb)

    Np2 = _ceil_to(N, 512)
    loops = jnp.arange(N, dtype=edge_index.dtype)
    row = jnp.concatenate([edge_index[0], loops])
    col = jnp.concatenate([edge_index[1], loops])

    Ed = _ceil_to(E, 4096)
    rowd = jnp.full((Ed,), Np2, jnp.int32).at[:E].set(edge_index[0])
    degp = _sc_deg(rowd, Np2)

    padv = (jnp.arange(Ep - EE, dtype=jnp.int32) % N)
    rowp = jnp.concatenate([row, padv])
    colp = jnp.concatenate([col, padv])
    pospad = jnp.zeros((Np2, 3), jnp.float32).at[:N].set(pos)
    msgp = _sc_msg(rowp, colp, pospad[:, 0], pospad[:, 1], pospad[:, 2],
                   degp).reshape(Ep, 8)

    h = _run_mlp1(msgp, p_local1, Eb)
    x = jax.ops.segment_max(h[:EE], col, num_segments=N)

    xp = jnp.zeros((Np2, x.shape[1]), jnp.float32).at[:N].set(x)
    xjp = _sc_gather_rows(xp, rowp)
    h2 = _run_mlp2(xjp, msgp, p_local2, Eb)
    x2 = jax.ops.segment_max(h2[:EE], col, num_segments=N)

    x2p = jnp.full((Np, x2.shape[1]), 0.0, jnp.float32).at[:N].set(x2)
    posp = jnp.zeros((Np, 8), jnp.float32).at[:N, :3].set(pos)
    batchp = jnp.full((Np, 8), 8, jnp.int32).at[:N, :].set(batch[:, None])

    out = _run_mlp3(x2p, posp, batchp, p_global, p_head, Nb)
    return out


# speedup vs baseline: 2.9226x; 1.7853x over previous
import functools

import jax
import jax.numpy as jnp
from jax import lax
from jax.experimental import pallas as pl
from jax.experimental.pallas import tpu as pltpu
from jax.experimental.pallas import tpu_sc as plsc

_EPS = 1e-5
_NC, _NS, _L = 2, 16, 16
_NW = _NC * _NS


def _sc_mesh():
    return plsc.VectorSubcoreMesh(core_axis_name="c", subcore_axis_name="s")


_SC_PARAMS = pltpu.CompilerParams(needs_layout_passes=False)


def _wid():
    return lax.axis_index("s") * _NC + lax.axis_index("c")


def _iota16():
    return lax.iota(jnp.int32, _L)


def _sc_deg(rowd, Np):
    Ed = rowd.shape[0]
    S = 4096
    Cn = Np // _NW

    @functools.partial(
        pl.kernel,
        out_type=jax.ShapeDtypeStruct((Np,), jnp.float32),
        mesh=_sc_mesh(),
        compiler_params=_SC_PARAMS,
        scratch_types=[pltpu.VMEM((Cn,), jnp.float32),
                       pltpu.VMEM((S,), jnp.int32)],
    )
    def k(row_hbm, deg_hbm, ldeg, rbuf):
        wid = _wid()
        lo = wid * Cn
        ones = jnp.ones((_L,), jnp.float32)

        def init(i, _):
            ldeg[pl.ds(i * _L, _L)] = ones
            return 0
        lax.fori_loop(0, Cn // _L, init, 0)

        def chunk(ci, _):
            pltpu.sync_copy(row_hbm.at[pl.ds(ci * S, S)], rbuf)

            def grp(g, _):
                r = rbuf[pl.ds(g * _L, _L)]
                msk = (r >= lo) & (r < lo + Cn)
                plsc.addupdate_scatter(ldeg, [r - lo], ones, mask=msk)
                return 0
            lax.fori_loop(0, S // _L, grp, 0)
            return 0
        lax.fori_loop(0, Ed // S, chunk, 0)
        pltpu.sync_copy(ldeg, deg_hbm.at[pl.ds(lo, Cn)])

    return k(rowd)


def _sc_msg(rowp, colp, px, py, pz, deg):
    Ep = rowp.shape[0]
    Np = px.shape[0]
    Te = Ep // _NW
    S = 672 if Te % 672 == 0 else 16
    nch = Te // S

    @functools.partial(
        pl.kernel,
        out_type=jax.ShapeDtypeStruct((Ep * 8,), jnp.float32),
        mesh=_sc_mesh(),
        compiler_params=_SC_PARAMS,
        scratch_types=[pltpu.VMEM((Np,), jnp.float32),
                       pltpu.VMEM((Np,), jnp.float32),
                       pltpu.VMEM((Np,), jnp.float32),
                       pltpu.VMEM((Np,), jnp.float32),
                       pltpu.VMEM((S,), jnp.int32),
                       pltpu.VMEM((S,), jnp.int32),
                       pltpu.VMEM((S * 8,), jnp.float32)],
    )
    def k(row_hbm, col_hbm, px_hbm, py_hbm, pz_hbm, deg_hbm, msg_hbm,
          pxv, pyv, pzv, dgv, rbuf, cbuf, obuf):
        wid = _wid()
        base0 = wid * Te
        pltpu.sync_copy(px_hbm, pxv)
        pltpu.sync_copy(py_hbm, pyv)
        pltpu.sync_copy(pz_hbm, pzv)
        pltpu.sync_copy(deg_hbm, dgv)
        zz = jnp.zeros((_L,), jnp.float32)

        def zinit(i, _):
            obuf[pl.ds(i * _L, _L)] = zz
            return 0
        lax.fori_loop(0, S * 8 // _L, zinit, 0)

        def chunk(ci, _):
            b = base0 + ci * S
            pltpu.sync_copy(row_hbm.at[pl.ds(b, S)], rbuf)
            pltpu.sync_copy(col_hbm.at[pl.ds(b, S)], cbuf)

            def grp(g, _):
                r = rbuf[pl.ds(g * _L, _L)]
                c = cbuf[pl.ds(g * _L, _L)]
                ew = 1.0 / plsc.load_gather(dgv, [r])
                dx = plsc.load_gather(pxv, [r]) - plsc.load_gather(pxv, [c])
                dy = plsc.load_gather(pyv, [r]) - plsc.load_gather(pyv, [c])
                dz = plsc.load_gather(pzv, [r]) - plsc.load_gather(pzv, [c])
                bi = (g * _L + _iota16()) * 8
                plsc.store_scatter(obuf, [bi], ew * dx)
                plsc.store_scatter(obuf, [bi + 1], ew * dy)
                plsc.store_scatter(obuf, [bi + 2], ew * dz)
                return 0
            lax.fori_loop(0, S // _L, grp, 0)
            pltpu.sync_copy(obuf, msg_hbm.at[pl.ds(b * 8, S * 8)])
            return 0
        lax.fori_loop(0, nch, chunk, 0)

    return k(rowp, colp, px, py, pz, deg)


def _sc_gather_rows(x, rowp):
    Np, F = x.shape
    Ep = rowp.shape[0]
    Te = Ep // _NW
    K = 128
    nch = Te // K

    @functools.partial(
        pl.kernel,
        out_type=jax.ShapeDtypeStruct((Ep, F), jnp.float32),
        mesh=_sc_mesh(),
        compiler_params=_SC_PARAMS,
        scratch_types=[pltpu.VMEM((K,), jnp.int32),
                       pltpu.VMEM((K, F), jnp.float32),
                       pltpu.SemaphoreType.DMA],
    )
    def k(x_hbm, row_hbm, out_hbm, idxv, gbuf, sem):
        wid = _wid()
        base0 = wid * Te

        def chunk(ci, _):
            b = base0 + ci * K
            pltpu.sync_copy(row_hbm.at[pl.ds(b, K)], idxv)
            pltpu.async_copy(x_hbm.at[idxv], gbuf, sem).wait()
            pltpu.sync_copy(gbuf, out_hbm.at[pl.ds(b, K)])
            return 0
        lax.fori_loop(0, nch, chunk, 0)

    return k(x, rowp)


def _sc_bucket(colb, Np):
    Ep = colb.shape[0]
    Cn = Np // _NW
    S = 4096
    FL = 4096
    VB = 2 * FL + _L
    IDCAP = _ceil_to(Ep, FL) + 2 * FL
    nch = (Ep + S - 1) // S

    @functools.partial(
        pl.kernel,
        out_type=(jax.ShapeDtypeStruct((_NW * IDCAP,), jnp.int32),
                  jax.ShapeDtypeStruct((_NW * IDCAP,), jnp.int32),
                  jax.ShapeDtypeStruct((_NW * 16,), jnp.int32)),
        mesh=_sc_mesh(),
        compiler_params=_SC_PARAMS,
        scratch_types=[pltpu.VMEM((S,), jnp.int32),
                       pltpu.VMEM((VB,), jnp.int32),
                       pltpu.VMEM((VB,), jnp.int32),
                       pltpu.VMEM((16,), jnp.int32)],
    )
    def k(col_hbm, ids_hbm, cls_hbm, cnt_hbm, cbuf, ebuf, lbuf, tmpv):
        wid = _wid()
        lo = wid * Cn

        def chunk(ci, carry):
            fill, outoff = carry
            pltpu.sync_copy(col_hbm.at[pl.ds(ci * S, S)], cbuf)

            def grp(g, fill):
                c = cbuf[pl.ds(g * _L, _L)]
                msk = (c >= lo) & (c < lo + Cn)
                e = ci * S + g * _L + _iota16()
                plsc.store_compressed(ebuf.at[pl.ds(fill, _L)], e, mask=msk)
                plsc.store_compressed(lbuf.at[pl.ds(fill, _L)], c - lo, mask=msk)
                pc = jnp.max(plsc.all_reduce_population_count(msk))
                return fill + pc
            fill = lax.fori_loop(0, S // _L, grp, fill)

            def do_flush(args):
                fill, outoff = args
                oo = wid * IDCAP + pl.multiple_of(outoff, FL)
                pltpu.sync_copy(ebuf.at[pl.ds(0, FL)],
                                ids_hbm.at[pl.ds(oo, FL)])
                pltpu.sync_copy(lbuf.at[pl.ds(0, FL)],
                                cls_hbm.at[pl.ds(oo, FL)])
                ngrp = (fill - FL + _L - 1) // _L

                def shift(i, _):
                    ev = ebuf[pl.ds(FL + i * _L, _L)]
                    lv = lbuf[pl.ds(FL + i * _L, _L)]
                    ebuf[pl.ds(i * _L, _L)] = ev
                    lbuf[pl.ds(i * _L, _L)] = lv
                    return 0
                lax.fori_loop(0, ngrp, shift, 0)
                return fill - FL, outoff + FL

            fill, outoff = lax.cond(fill >= FL, do_flush, lambda a: a,
                                    (fill, outoff))
            return fill, outoff
        fill, outoff = lax.fori_loop(0, nch, chunk,
                                     (jnp.int32(0), jnp.int32(0)))
        oo = wid * IDCAP + pl.multiple_of(outoff, FL)
        pltpu.sync_copy(ebuf.at[pl.ds(0, FL)], ids_hbm.at[pl.ds(oo, FL)])
        pltpu.sync_copy(lbuf.at[pl.ds(0, FL)], cls_hbm.at[pl.ds(oo, FL)])
        tmpv[...] = jnp.where(_iota16() == 0, fill + outoff, 0)
        pltpu.sync_copy(tmpv, cnt_hbm.at[pl.ds(wid * 16, 16)])

    return k(colb)


def _sc_scatter_max(h, ids, cls, cnt, Np, HB):
    Ep, D = h.shape
    Cn = Np // _NW
    nfc = D // _L
    SC_E = 4096
    nsg = SC_E // _L
    IDCAP = ids.shape[0] // _NW

    @functools.partial(
        pl.kernel,
        out_type=jax.ShapeDtypeStruct((Np, D), jnp.float32),
        mesh=_sc_mesh(),
        compiler_params=_SC_PARAMS,
        scratch_types=[pltpu.VMEM((Cn + 8, D), jnp.float32),
                       pltpu.VMEM((SC_E,), jnp.int32),
                       pltpu.VMEM((SC_E,), jnp.int32),
                       pltpu.VMEM((HB, D), jnp.float32),
                       pltpu.VMEM((HB, D), jnp.float32),
                       pltpu.VMEM((16,), jnp.int32),
                       pltpu.SemaphoreType.DMA,
                       pltpu.SemaphoreType.DMA],
    )
    def k(h_hbm, ids_hbm, cls_hbm, cnt_hbm, out_hbm,
          tbl, idsv, clsv, hb0, hb1, cntv, sem0, sem1):
        wid = _wid()
        lo = wid * Cn
        hbufs = (hb0, hb1)
        sems = (sem0, sem1)
        neg = jnp.full((_L,), -jnp.inf, jnp.float32)

        def init(r, _):
            for j in range(nfc):
                tbl[r, pl.ds(j * _L, _L)] = neg
            return 0
        lax.fori_loop(0, Cn + 8, init, 0)

        pltpu.sync_copy(cnt_hbm.at[pl.ds(wid * 16, 16)], cntv)
        n = jnp.max(jnp.where(_iota16() == 0, cntv[...], 0))

        def fire(si, b):
            pltpu.async_copy(h_hbm.at[idsv.at[pl.ds(si * HB, HB)]],
                             hbufs[b], sems[b])

        def wait(b):
            pltpu.make_async_copy(h_hbm.at[pl.ds(0, HB)], hbufs[b],
                                  sems[b]).wait()

        def process(si, b):
            hbuf = hbufs[b]

            def grp(g, _):
                for lane in range(_L):
                    ent = si * HB + g * _L + lane
                    spl = jnp.zeros((_L,), jnp.int32) + ent
                    cspl = plsc.load_gather(clsv, [spl])
                    for j in range(nfc):
                        hv = hbuf[g * _L + lane, pl.ds(j * _L, _L)]
                        jv = j * _L + _iota16()
                        tv = plsc.load_gather(tbl, [cspl, jv])
                        plsc.store_scatter(tbl, [cspl, jv], jnp.maximum(tv, hv))
                return 0
            lax.fori_loop(0, HB // _L, grp, 0)

        nsc = (n + SC_E - 1) // SC_E

        def super_chunk(sc, _):
            sb = sc * SC_E
            sbo = wid * IDCAP + sb
            pltpu.sync_copy(ids_hbm.at[pl.ds(sbo, SC_E)], idsv)
            pltpu.sync_copy(cls_hbm.at[pl.ds(sbo, SC_E)], clsv)

            def sani(gg, _):
                pos = sb + gg * _L + _iota16()
                valid = pos < n
                ev = idsv[pl.ds(gg * _L, _L)]
                cv = clsv[pl.ds(gg * _L, _L)]
                idsv[pl.ds(gg * _L, _L)] = jnp.where(valid, ev,
                                                     pos - sb)
                clsv[pl.ds(gg * _L, _L)] = jnp.where(valid, cv, Cn)
                return 0
            lax.fori_loop(0, nsg, sani, 0)

            msub = (jnp.minimum(SC_E, n - sb) + HB - 1) // HB

            fire(0, 0)

            def dbl(kk, _):
                si0 = 2 * kk
                si1 = 2 * kk + 1

                @pl.when(si1 < msub)
                def _():
                    fire(si1, 1)
                wait(0)
                process(si0, 0)

                @pl.when(si0 + 2 < msub)
                def _():
                    fire(si0 + 2, 0)

                @pl.when(si1 < msub)
                def _():
                    wait(1)
                    process(si1, 1)
                return 0
            lax.fori_loop(0, (msub + 1) // 2, dbl, 0)
            return 0
        lax.fori_loop(0, nsc, super_chunk, 0)

        pltpu.sync_copy(tbl.at[pl.ds(0, Cn)], out_hbm.at[pl.ds(lo, Cn)])

    return k(h, ids, cls, cnt)


def _ln(x, g, b):
    mu = jnp.mean(x, axis=-1, keepdims=True)
    var = jnp.mean((x - mu) ** 2, axis=-1, keepdims=True)
    return (x - mu) * lax.rsqrt(var + _EPS) * g + b


def _relu(x):
    return jnp.maximum(x, 0.0)


def _mlp1_body(msg_ref, g1, b1, W1, g2, b2, W2, g3, b3, W3, out_ref):
    m = msg_ref[...]
    mask3 = (lax.broadcasted_iota(jnp.int32, m.shape, 1) < 3).astype(jnp.float32)
    mu = jnp.sum(m, axis=1, keepdims=True) * (1.0 / 3.0)
    var = jnp.sum(((m - mu) ** 2) * mask3, axis=1, keepdims=True) * (1.0 / 3.0)
    a = _relu((m - mu) * lax.rsqrt(var + _EPS) * g1[...] + b1[...])
    h = jnp.dot(a, W1[...], preferred_element_type=jnp.float32)
    h = jnp.dot(_relu(_ln(h, g2[...], b2[...])), W2[...], preferred_element_type=jnp.float32)
    h = jnp.dot(_relu(_ln(h, g3[...], b3[...])), W3[...], preferred_element_type=jnp.float32)
    out_ref[...] = h


def _run_mlp1(msgp, p1, Eb):
    Ep = msgp.shape[0]
    (g1, b1, W1), (g2, b2, W2), (g3, b3, W3) = p1
    g1p = jnp.concatenate([g1, jnp.ones((5,), jnp.float32)]).reshape(1, 8)
    b1p = jnp.concatenate([b1, jnp.zeros((5,), jnp.float32)]).reshape(1, 8)
    W1p = jnp.concatenate([W1, jnp.zeros((5, W1.shape[1]), jnp.float32)], axis=0)
    d1, d2, d3 = W1.shape[1], W2.shape[1], W3.shape[1]
    grid = (Ep // Eb,)
    full = lambda shape: pl.BlockSpec(shape, lambda i: (0, 0))
    return pl.pallas_call(
        _mlp1_body,
        grid=grid,
        in_specs=[
            pl.BlockSpec((Eb, 8), lambda i: (i, 0)),
            full((1, 8)), full((1, 8)), full((8, d1)),
            full((1, d1)), full((1, d1)), full((d1, d2)),
            full((1, d2)), full((1, d2)), full((d2, d3)),
        ],
        out_specs=pl.BlockSpec((Eb, d3), lambda i: (i, 0)),
        out_shape=jax.ShapeDtypeStruct((Ep, d3), jnp.float32),
    )(msgp, g1p, b1p, W1p, g2.reshape(1, -1), b2.reshape(1, -1), W2,
      g3.reshape(1, -1), b3.reshape(1, -1), W3)


def _mlp2_body(xj_ref, msg_ref, g1x, b1x, g1m, b1m, W1a, W1b,
               g2, b2, W2, g3, b3, W3, out_ref):
    xj = xj_ref[...]
    m = msg_ref[...]
    C = xj.shape[1] + 3.0
    mask3 = (lax.broadcasted_iota(jnp.int32, m.shape, 1) < 3).astype(jnp.float32)
    s = jnp.sum(xj, axis=1, keepdims=True) + jnp.sum(m, axis=1, keepdims=True)
    mu = s / C
    var = (jnp.sum((xj - mu) ** 2, axis=1, keepdims=True)
           + jnp.sum(((m - mu) ** 2) * mask3, axis=1, keepdims=True)) / C
    inv = lax.rsqrt(var + _EPS)
    ax = _relu((xj - mu) * inv * g1x[...] + b1x[...])
    am = _relu((m - mu) * inv * g1m[...] + b1m[...])
    h = (jnp.dot(ax, W1a[...], preferred_element_type=jnp.float32)
         + jnp.dot(am, W1b[...], preferred_element_type=jnp.float32))
    h = jnp.dot(_relu(_ln(h, g2[...], b2[...])), W2[...], preferred_element_type=jnp.float32)
    h = jnp.dot(_relu(_ln(h, g3[...], b3[...])), W3[...], preferred_element_type=jnp.float32)
    out_ref[...] = h


def _run_mlp2(xjp, msgp, p2, Eb):
    Ep, F = xjp.shape
    (g1, b1, W1), (g2, b2, W2), (g3, b3, W3) = p2
    g1x, g1m = g1[:F].reshape(1, F), jnp.concatenate(
        [g1[F:], jnp.ones((5,), jnp.float32)]).reshape(1, 8)
    b1x, b1m = b1[:F].reshape(1, F), jnp.concatenate(
        [b1[F:], jnp.zeros((5,), jnp.float32)]).reshape(1, 8)
    W1a = W1[:F]
    W1b = jnp.concatenate([W1[F:], jnp.zeros((5, W1.shape[1]), jnp.float32)], axis=0)
    d1, d2, d3 = W1.shape[1], W2.shape[1], W3.shape[1]
    grid = (Ep // Eb,)
    full = lambda shape: pl.BlockSpec(shape, lambda i: (0, 0))
    return pl.pallas_call(
        _mlp2_body,
        grid=grid,
        in_specs=[
            pl.BlockSpec((Eb, F), lambda i: (i, 0)),
            pl.BlockSpec((Eb, 8), lambda i: (i, 0)),
            full((1, F)), full((1, F)), full((1, 8)), full((1, 8)),
            full((F, d1)), full((8, d1)),
            full((1, d1)), full((1, d1)), full((d1, d2)),
            full((1, d2)), full((1, d2)), full((d2, d3)),
        ],
        out_specs=pl.BlockSpec((Eb, d3), lambda i: (i, 0)),
        out_shape=jax.ShapeDtypeStruct((Ep, d3), jnp.float32),
    )(xjp, msgp, g1x, b1x, g1m, b1m, W1a, W1b,
      g2.reshape(1, -1), b2.reshape(1, -1), W2,
      g3.reshape(1, -1), b3.reshape(1, -1), W3)


def _mlp3_body(x2_ref, pos_ref, batch_ref,
               g1x, b1x, g1p, b1p, W1a, W1b, g2, b2, W2, g3, b3, W3,
               hg1, hb1, HW1, hc1, hg2, hb2, HW2, hc2, hg3, hb3, HW3, hc3,
               out_ref, pooled_ref):
    i = pl.program_id(0)
    nblk = pl.num_programs(0)

    @pl.when(i == 0)
    def _():
        pooled_ref[...] = jnp.full_like(pooled_ref, -jnp.inf)

    x2 = x2_ref[...]
    p = pos_ref[...]
    C = x2.shape[1] + 3.0
    mask3 = (lax.broadcasted_iota(jnp.int32, p.shape, 1) < 3).astype(jnp.float32)
    mu = (jnp.sum(x2, axis=1, keepdims=True) + jnp.sum(p, axis=1, keepdims=True)) / C
    var = (jnp.sum((x2 - mu) ** 2, axis=1, keepdims=True)
           + jnp.sum(((p - mu) ** 2) * mask3, axis=1, keepdims=True)) / C
    inv = lax.rsqrt(var + _EPS)
    ax = _relu((x2 - mu) * inv * g1x[...] + b1x[...])
    ap = _relu((p - mu) * inv * g1p[...] + b1p[...])
    h = (jnp.dot(ax, W1a[...], preferred_element_type=jnp.float32)
         + jnp.dot(ap, W1b[...], preferred_element_type=jnp.float32))
    h = jnp.dot(_relu(_ln(h, g2[...], b2[...])), W2[...], preferred_element_type=jnp.float32)
    g_out = jnp.dot(_relu(_ln(h, g3[...], b3[...])), W3[...], preferred_element_type=jnp.float32)

    b0 = batch_ref[...][:, :1]
    for bb in range(8):
        mask = b0 == bb
        contrib = jnp.max(jnp.where(mask, g_out, -jnp.inf), axis=0, keepdims=True)
        pooled_ref[bb:bb + 1, :] = jnp.maximum(pooled_ref[bb:bb + 1, :], contrib)

    @pl.when(i == nblk - 1)
    def _():
        pool = pooled_ref[...]
        o = jnp.dot(_relu(_ln(pool, hg1[...], hb1[...])), HW1[...],
                    preferred_element_type=jnp.float32) + hc1[...]
        o = jnp.dot(_relu(_ln(o, hg2[...], hb2[...])), HW2[...],
                    preferred_element_type=jnp.float32) + hc2[...]
        o = jnp.dot(_relu(_ln(o, hg3[...], hb3[...])), HW3[...],
                    preferred_element_type=jnp.float32) + hc3[...]
        out_ref[...] = o


def _run_mlp3(x2p, posp, batchp, pg, ph, Nb):
    Np, F2 = x2p.shape
    (g1, b1, W1), (g2, b2, W2), (g3, b3, W3) = pg
    (hg1, hb1, HW1, hc1), (hg2, hb2, HW2, hc2), (hg3, hb3, HW3, hc3) = ph
    g1x, g1p = g1[:F2].reshape(1, F2), jnp.concatenate(
        [g1[F2:], jnp.ones((5,), jnp.float32)]).reshape(1, 8)
    b1x, b1p = b1[:F2].reshape(1, F2), jnp.concatenate(
        [b1[F2:], jnp.zeros((5,), jnp.float32)]).reshape(1, 8)
    W1a = W1[:F2]
    W1b = jnp.concatenate([W1[F2:], jnp.zeros((5, W1.shape[1]), jnp.float32)], axis=0)
    d1, d2, d3 = W1.shape[1], W2.shape[1], W3.shape[1]
    e1, e2, e3 = HW1.shape[1], HW2.shape[1], HW3.shape[1]
    grid = (Np // Nb,)
    full = lambda shape: pl.BlockSpec(shape, lambda i: (0, 0))
    r2 = lambda v: v.reshape(1, -1)
    return pl.pallas_call(
        _mlp3_body,
        grid=grid,
        in_specs=[
            pl.BlockSpec((Nb, F2), lambda i: (i, 0)),
            pl.BlockSpec((Nb, 8), lambda i: (i, 0)),
            pl.BlockSpec((Nb, 8), lambda i: (i, 0)),
            full((1, F2)), full((1, F2)), full((1, 8)), full((1, 8)),
            full((F2, d1)), full((8, d1)),
            full((1, d1)), full((1, d1)), full((d1, d2)),
            full((1, d2)), full((1, d2)), full((d2, d3)),
            full((1, d3)), full((1, d3)), full((d3, e1)), full((1, e1)),
            full((1, e1)), full((1, e1)), full((e1, e2)), full((1, e2)),
            full((1, e2)), full((1, e2)), full((e2, e3)), full((1, e3)),
        ],
        out_specs=pl.BlockSpec((8, e3), lambda i: (0, 0)),
        out_shape=jax.ShapeDtypeStruct((8, e3), jnp.float32),
        scratch_shapes=[pltpu.VMEM((8, d3), jnp.float32)],
    )(x2p, posp, batchp, g1x, b1x, g1p, b1p, W1a, W1b,
      r2(g2), r2(b2), W2, r2(g3), r2(b3), W3,
      r2(hg1), r2(hb1), HW1, r2(hc1), r2(hg2), r2(hb2), HW2, r2(hc2),
      r2(hg3), r2(hb3), HW3, r2(hc3))


def _ceil_to(x, m):
    return (x + m - 1) // m * m


def kernel(pos, edge_index, batch, p_local1, p_local2, p_global, p_head):
    N = pos.shape[0]
    E = edge_index.shape[1]
    EE = E + N
    Eb = 2048
    Nb = 512
    Ep = _ceil_to(EE, Eb)
    Np = _ceil_to(N, 512)
    Np2 = Np
    loops = jnp.arange(N, dtype=edge_index.dtype)
    row = jnp.concatenate([edge_index[0], loops])
    col = jnp.concatenate([edge_index[1], loops])

    Ed = _ceil_to(E, 4096)
    rowd = jnp.full((Ed,), Np2, jnp.int32).at[:E].set(edge_index[0])
    degp = _sc_deg(rowd, Np2)

    padv = (jnp.arange(Ep - EE, dtype=jnp.int32) % N)
    rowp = jnp.concatenate([row, padv])
    colp = jnp.concatenate([col, padv])
    pospad = jnp.zeros((Np2, 3), jnp.float32).at[:N].set(pos)
    msgp = _sc_msg(rowp, colp, pospad[:, 0], pospad[:, 1], pospad[:, 2],
                   degp).reshape(Ep, 8)

    colb = jnp.concatenate([col, jnp.full((Ep - EE,), Np2, jnp.int32)])
    ids, cls_, cnt = _sc_bucket(colb, Np2)

    h = _run_mlp1(msgp, p_local1, Eb)
    x = _sc_scatter_max(h, ids, cls_, cnt, Np2, 128)

    xjp = _sc_gather_rows(x, rowp)
    h2 = _run_mlp2(xjp, msgp, p_local2, Eb)
    x2 = _sc_scatter_max(h2, ids, cls_, cnt, Np2, 64)

    posp = jnp.zeros((Np, 8), jnp.float32).at[:N, :3].set(pos)
    batchp = jnp.full((Np, 8), 8, jnp.int32).at[:N, :].set(batch[:, None])

    out = _run_mlp3(x2, posp, batchp, p_global, p_head, Nb)
    return out


# speedup vs baseline: 4.5774x; 1.5662x over previous
import functools

import jax
import jax.numpy as jnp
from jax import lax
from jax.experimental import pallas as pl
from jax.experimental.pallas import tpu as pltpu
from jax.experimental.pallas import tpu_sc as plsc

_EPS = 1e-5
_NC, _NS, _L = 2, 16, 16
_NW = _NC * _NS


def _sc_mesh():
    return plsc.VectorSubcoreMesh(core_axis_name="c", subcore_axis_name="s")


_SC_PARAMS = pltpu.CompilerParams(needs_layout_passes=False)


def _wid():
    return lax.axis_index("s") * _NC + lax.axis_index("c")


def _iota16():
    return lax.iota(jnp.int32, _L)


def _sc_deg(rowd, Np):
    Ed = rowd.shape[0]
    S = 4096
    Cn = Np // _NW

    @functools.partial(
        pl.kernel,
        out_type=jax.ShapeDtypeStruct((Np,), jnp.float32),
        mesh=_sc_mesh(),
        compiler_params=_SC_PARAMS,
        scratch_types=[pltpu.VMEM((Cn,), jnp.float32),
                       pltpu.VMEM((S,), jnp.int32)],
    )
    def k(row_hbm, deg_hbm, ldeg, rbuf):
        wid = _wid()
        lo = wid * Cn
        ones = jnp.ones((_L,), jnp.float32)

        def init(i, _):
            ldeg[pl.ds(i * _L, _L)] = ones
            return 0
        lax.fori_loop(0, Cn // _L, init, 0)

        def chunk(ci, _):
            pltpu.sync_copy(row_hbm.at[pl.ds(ci * S, S)], rbuf)

            def grp(g, _):
                r = rbuf[pl.ds(g * _L, _L)]
                msk = (r >= lo) & (r < lo + Cn)
                plsc.addupdate_scatter(ldeg, [r - lo], ones, mask=msk)
                return 0
            lax.fori_loop(0, S // _L, grp, 0)
            return 0
        lax.fori_loop(0, Ed // S, chunk, 0)
        pltpu.sync_copy(ldeg, deg_hbm.at[pl.ds(lo, Cn)])

    return k(rowd)


def _sc_msg(rowp, colp, px, py, pz, deg):
    Ep = rowp.shape[0]
    Np = px.shape[0]
    Te = Ep // _NW
    S = 672 if Te % 672 == 0 else 16
    nch = Te // S

    @functools.partial(
        pl.kernel,
        out_type=jax.ShapeDtypeStruct((Ep * 8,), jnp.float32),
        mesh=_sc_mesh(),
        compiler_params=_SC_PARAMS,
        scratch_types=[pltpu.VMEM((Np,), jnp.float32),
                       pltpu.VMEM((Np,), jnp.float32),
                       pltpu.VMEM((Np,), jnp.float32),
                       pltpu.VMEM((Np,), jnp.float32),
                       pltpu.VMEM((S,), jnp.int32),
                       pltpu.VMEM((S,), jnp.int32),
                       pltpu.VMEM((S * 8,), jnp.float32)],
    )
    def k(row_hbm, col_hbm, px_hbm, py_hbm, pz_hbm, deg_hbm, msg_hbm,
          pxv, pyv, pzv, dgv, rbuf, cbuf, obuf):
        wid = _wid()
        base0 = wid * Te
        pltpu.sync_copy(px_hbm, pxv)
        pltpu.sync_copy(py_hbm, pyv)
        pltpu.sync_copy(pz_hbm, pzv)
        pltpu.sync_copy(deg_hbm, dgv)
        zz = jnp.zeros((_L,), jnp.float32)

        def zinit(i, _):
            obuf[pl.ds(i * _L, _L)] = zz
            return 0
        lax.fori_loop(0, S * 8 // _L, zinit, 0)

        def chunk(ci, _):
            b = base0 + ci * S
            pltpu.sync_copy(row_hbm.at[pl.ds(b, S)], rbuf)
            pltpu.sync_copy(col_hbm.at[pl.ds(b, S)], cbuf)

            def grp(g, _):
                r = rbuf[pl.ds(g * _L, _L)]
                c = cbuf[pl.ds(g * _L, _L)]
                ew = 1.0 / plsc.load_gather(dgv, [r])
                dx = plsc.load_gather(pxv, [r]) - plsc.load_gather(pxv, [c])
                dy = plsc.load_gather(pyv, [r]) - plsc.load_gather(pyv, [c])
                dz = plsc.load_gather(pzv, [r]) - plsc.load_gather(pzv, [c])
                bi = (g * _L + _iota16()) * 8
                plsc.store_scatter(obuf, [bi], ew * dx)
                plsc.store_scatter(obuf, [bi + 1], ew * dy)
                plsc.store_scatter(obuf, [bi + 2], ew * dz)
                return 0
            lax.fori_loop(0, S // _L, grp, 0)
            pltpu.sync_copy(obuf, msg_hbm.at[pl.ds(b * 8, S * 8)])
            return 0
        lax.fori_loop(0, nch, chunk, 0)

    return k(rowp, colp, px, py, pz, deg)


def _sc_gather_rows(x, rowp):
    Np, F = x.shape
    Ep = rowp.shape[0]
    Te = Ep // _NW
    K = 128
    nch = Te // K

    @functools.partial(
        pl.kernel,
        out_type=jax.ShapeDtypeStruct((Ep, F), jnp.float32),
        mesh=_sc_mesh(),
        compiler_params=_SC_PARAMS,
        scratch_types=[pltpu.VMEM((K,), jnp.int32),
                       pltpu.VMEM((K, F), jnp.float32),
                       pltpu.SemaphoreType.DMA],
    )
    def k(x_hbm, row_hbm, out_hbm, idxv, gbuf, sem):
        wid = _wid()
        base0 = wid * Te

        def chunk(ci, _):
            b = base0 + ci * K
            pltpu.sync_copy(row_hbm.at[pl.ds(b, K)], idxv)
            pltpu.async_copy(x_hbm.at[idxv], gbuf, sem).wait()
            pltpu.sync_copy(gbuf, out_hbm.at[pl.ds(b, K)])
            return 0
        lax.fori_loop(0, nch, chunk, 0)

    return k(x, rowp)


def _sc_bucket(colb, Np):
    Ep = colb.shape[0]
    Cn = Np // _NW
    S = 4096
    FL = 4096
    VB = 2 * FL + _L
    IDCAP = _ceil_to(Ep, FL) + 2 * FL
    nch = (Ep + S - 1) // S

    @functools.partial(
        pl.kernel,
        out_type=(jax.ShapeDtypeStruct((_NW * IDCAP,), jnp.int32),
                  jax.ShapeDtypeStruct((_NW * IDCAP,), jnp.int32),
                  jax.ShapeDtypeStruct((_NW * 16,), jnp.int32)),
        mesh=_sc_mesh(),
        compiler_params=_SC_PARAMS,
        scratch_types=[pltpu.VMEM((S,), jnp.int32),
                       pltpu.VMEM((VB,), jnp.int32),
                       pltpu.VMEM((VB,), jnp.int32),
                       pltpu.VMEM((16,), jnp.int32)],
    )
    def k(col_hbm, ids_hbm, cls_hbm, cnt_hbm, cbuf, ebuf, lbuf, tmpv):
        wid = _wid()
        lo = wid * Cn

        def chunk(ci, carry):
            fill, outoff = carry
            pltpu.sync_copy(col_hbm.at[pl.ds(ci * S, S)], cbuf)

            def grp(g, fill):
                c = cbuf[pl.ds(g * _L, _L)]
                msk = (c >= lo) & (c < lo + Cn)
                e = ci * S + g * _L + _iota16()
                plsc.store_compressed(ebuf.at[pl.ds(fill, _L)], e, mask=msk)
                plsc.store_compressed(lbuf.at[pl.ds(fill, _L)], c - lo, mask=msk)
                pc = jnp.max(plsc.all_reduce_population_count(msk))
                return fill + pc
            fill = lax.fori_loop(0, S // _L, grp, fill)

            def do_flush(args):
                fill, outoff = args
                oo = wid * IDCAP + pl.multiple_of(outoff, FL)
                pltpu.sync_copy(ebuf.at[pl.ds(0, FL)],
                                ids_hbm.at[pl.ds(oo, FL)])
                pltpu.sync_copy(lbuf.at[pl.ds(0, FL)],
                                cls_hbm.at[pl.ds(oo, FL)])
                ngrp = (fill - FL + _L - 1) // _L

                def shift(i, _):
                    ev = ebuf[pl.ds(FL + i * _L, _L)]
                    lv = lbuf[pl.ds(FL + i * _L, _L)]
                    ebuf[pl.ds(i * _L, _L)] = ev
                    lbuf[pl.ds(i * _L, _L)] = lv
                    return 0
                lax.fori_loop(0, ngrp, shift, 0)
                return fill - FL, outoff + FL

            fill, outoff = lax.cond(fill >= FL, do_flush, lambda a: a,
                                    (fill, outoff))
            return fill, outoff
        fill, outoff = lax.fori_loop(0, nch, chunk,
                                     (jnp.int32(0), jnp.int32(0)))
        oo = wid * IDCAP + pl.multiple_of(outoff, FL)
        pltpu.sync_copy(ebuf.at[pl.ds(0, FL)], ids_hbm.at[pl.ds(oo, FL)])
        pltpu.sync_copy(lbuf.at[pl.ds(0, FL)], cls_hbm.at[pl.ds(oo, FL)])
        tmpv[...] = jnp.where(_iota16() == 0, fill + outoff, 0)
        pltpu.sync_copy(tmpv, cnt_hbm.at[pl.ds(wid * 16, 16)])

    return k(colb)


def _sc_scatter_max(h, ids, cls, cnt, Np, HB):
    Ep, D = h.shape
    Cn = Np // _NW
    nfc = D // _L
    SC_E = 4096
    nsg = SC_E // _L
    IDCAP = ids.shape[0] // _NW

    @functools.partial(
        pl.kernel,
        out_type=jax.ShapeDtypeStruct((Np * D,), jnp.float32),
        mesh=_sc_mesh(),
        compiler_params=_SC_PARAMS,
        scratch_types=[pltpu.VMEM(((Cn + 8) * D,), jnp.float32),
                       pltpu.VMEM((SC_E,), jnp.int32),
                       pltpu.VMEM((SC_E,), jnp.int32),
                       pltpu.VMEM((HB, D), jnp.float32),
                       pltpu.VMEM((HB, D), jnp.float32),
                       pltpu.VMEM((16,), jnp.int32),
                       pltpu.SemaphoreType.DMA,
                       pltpu.SemaphoreType.DMA],
    )
    def k(h_hbm, ids_hbm, cls_hbm, cnt_hbm, out_hbm,
          tbl, idsv, clsv, hb0, hb1, cntv, sem0, sem1):
        wid = _wid()
        lo = wid * Cn
        hbufs = (hb0, hb1)
        sems = (sem0, sem1)
        neg = jnp.full((_L,), -jnp.inf, jnp.float32)

        def init(r, _):
            tbl[pl.ds(r * _L, _L)] = neg
            return 0
        lax.fori_loop(0, (Cn + 8) * D // _L, init, 0)

        pltpu.sync_copy(cnt_hbm.at[pl.ds(wid * 16, 16)], cntv)
        n = jnp.max(jnp.where(_iota16() == 0, cntv[...], 0))

        def fire(si, b):
            pltpu.async_copy(h_hbm.at[idsv.at[pl.ds(si * HB, HB)]],
                             hbufs[b], sems[b])

        def wait(b):
            pltpu.make_async_copy(h_hbm.at[pl.ds(0, HB)], hbufs[b],
                                  sems[b]).wait()

        def process(si, b):
            hbuf = hbufs[b]
            half = 8

            def grp(g, _):
                for lane in range(_L):
                    ent = si * HB + g * _L + lane
                    spl = jnp.zeros((_L,), jnp.int32) + ent
                    base = plsc.load_gather(clsv, [spl]) * D + _iota16()
                    for j0 in range(0, nfc, half):
                        avs, hvs, tvs = [], [], []
                        for j in range(j0, min(j0 + half, nfc)):
                            av = base + j * _L
                            avs.append(av)
                            hvs.append(hbuf[g * _L + lane, pl.ds(j * _L, _L)])
                            tvs.append(plsc.load_gather(tbl, [av]))
                        for av, hv, tv in zip(avs, hvs, tvs):
                            plsc.store_scatter(tbl, [av], jnp.maximum(tv, hv))
                return 0
            lax.fori_loop(0, HB // _L, grp, 0)

        nsc = (n + SC_E - 1) // SC_E

        def super_chunk(sc, _):
            sb = sc * SC_E
            sbo = wid * IDCAP + sb
            pltpu.sync_copy(ids_hbm.at[pl.ds(sbo, SC_E)], idsv)
            pltpu.sync_copy(cls_hbm.at[pl.ds(sbo, SC_E)], clsv)

            def sani(gg, _):
                pos = sb + gg * _L + _iota16()
                valid = pos < n
                ev = idsv[pl.ds(gg * _L, _L)]
                cv = clsv[pl.ds(gg * _L, _L)]
                idsv[pl.ds(gg * _L, _L)] = jnp.where(valid, ev,
                                                     pos - sb)
                clsv[pl.ds(gg * _L, _L)] = jnp.where(valid, cv, Cn)
                return 0
            lax.fori_loop(0, nsg, sani, 0)

            msub = (jnp.minimum(SC_E, n - sb) + HB - 1) // HB

            fire(0, 0)

            def dbl(kk, _):
                si0 = 2 * kk
                si1 = 2 * kk + 1

                @pl.when(si1 < msub)
                def _():
                    fire(si1, 1)
                wait(0)
                process(si0, 0)

                @pl.when(si0 + 2 < msub)
                def _():
                    fire(si0 + 2, 0)

                @pl.when(si1 < msub)
                def _():
                    wait(1)
                    process(si1, 1)
                return 0
            lax.fori_loop(0, (msub + 1) // 2, dbl, 0)
            return 0
        lax.fori_loop(0, nsc, super_chunk, 0)

        pltpu.sync_copy(tbl.at[pl.ds(0, Cn * D)],
                        out_hbm.at[pl.ds(lo * D, Cn * D)])

    return k(h, ids, cls, cnt)


def _ln(x, g, b):
    mu = jnp.mean(x, axis=-1, keepdims=True)
    var = jnp.mean((x - mu) ** 2, axis=-1, keepdims=True)
    return (x - mu) * lax.rsqrt(var + _EPS) * g + b


def _relu(x):
    return jnp.maximum(x, 0.0)


def _mlp1_body(msg_ref, g1, b1, W1, g2, b2, W2, g3, b3, W3, out_ref):
    m = msg_ref[...]
    mask3 = (lax.broadcasted_iota(jnp.int32, m.shape, 1) < 3).astype(jnp.float32)
    mu = jnp.sum(m, axis=1, keepdims=True) * (1.0 / 3.0)
    var = jnp.sum(((m - mu) ** 2) * mask3, axis=1, keepdims=True) * (1.0 / 3.0)
    a = _relu((m - mu) * lax.rsqrt(var + _EPS) * g1[...] + b1[...])
    h = jnp.dot(a, W1[...], preferred_element_type=jnp.float32)
    h = jnp.dot(_relu(_ln(h, g2[...], b2[...])), W2[...], preferred_element_type=jnp.float32)
    h = jnp.dot(_relu(_ln(h, g3[...], b3[...])), W3[...], preferred_element_type=jnp.float32)
    out_ref[...] = h


def _run_mlp1(msgp, p1, Eb):
    Ep = msgp.shape[0]
    (g1, b1, W1), (g2, b2, W2), (g3, b3, W3) = p1
    g1p = jnp.concatenate([g1, jnp.ones((5,), jnp.float32)]).reshape(1, 8)
    b1p = jnp.concatenate([b1, jnp.zeros((5,), jnp.float32)]).reshape(1, 8)
    W1p = jnp.concatenate([W1, jnp.zeros((5, W1.shape[1]), jnp.float32)], axis=0)
    d1, d2, d3 = W1.shape[1], W2.shape[1], W3.shape[1]
    grid = (Ep // Eb,)
    full = lambda shape: pl.BlockSpec(shape, lambda i: (0, 0))
    return pl.pallas_call(
        _mlp1_body,
        grid=grid,
        in_specs=[
            pl.BlockSpec((Eb, 8), lambda i: (i, 0)),
            full((1, 8)), full((1, 8)), full((8, d1)),
            full((1, d1)), full((1, d1)), full((d1, d2)),
            full((1, d2)), full((1, d2)), full((d2, d3)),
        ],
        out_specs=pl.BlockSpec((Eb, d3), lambda i: (i, 0)),
        out_shape=jax.ShapeDtypeStruct((Ep, d3), jnp.float32),
    )(msgp, g1p, b1p, W1p, g2.reshape(1, -1), b2.reshape(1, -1), W2,
      g3.reshape(1, -1), b3.reshape(1, -1), W3)


def _mlp2_body(xj_ref, msg_ref, g1x, b1x, g1m, b1m, W1a, W1b,
               g2, b2, W2, g3, b3, W3, out_ref):
    xj = xj_ref[...]
    m = msg_ref[...]
    C = xj.shape[1] + 3.0
    mask3 = (lax.broadcasted_iota(jnp.int32, m.shape, 1) < 3).astype(jnp.float32)
    s = jnp.sum(xj, axis=1, keepdims=True) + jnp.sum(m, axis=1, keepdims=True)
    mu = s / C
    var = (jnp.sum((xj - mu) ** 2, axis=1, keepdims=True)
           + jnp.sum(((m - mu) ** 2) * mask3, axis=1, keepdims=True)) / C
    inv = lax.rsqrt(var + _EPS)
    ax = _relu((xj - mu) * inv * g1x[...] + b1x[...])
    am = _relu((m - mu) * inv * g1m[...] + b1m[...])
    h = (jnp.dot(ax, W1a[...], preferred_element_type=jnp.float32)
         + jnp.dot(am, W1b[...], preferred_element_type=jnp.float32))
    h = jnp.dot(_relu(_ln(h, g2[...], b2[...])), W2[...], preferred_element_type=jnp.float32)
    h = jnp.dot(_relu(_ln(h, g3[...], b3[...])), W3[...], preferred_element_type=jnp.float32)
    out_ref[...] = h


def _run_mlp2(xjp, msgp, p2, Eb):
    Ep, F = xjp.shape
    (g1, b1, W1), (g2, b2, W2), (g3, b3, W3) = p2
    g1x, g1m = g1[:F].reshape(1, F), jnp.concatenate(
        [g1[F:], jnp.ones((5,), jnp.float32)]).reshape(1, 8)
    b1x, b1m = b1[:F].reshape(1, F), jnp.concatenate(
        [b1[F:], jnp.zeros((5,), jnp.float32)]).reshape(1, 8)
    W1a = W1[:F]
    W1b = jnp.concatenate([W1[F:], jnp.zeros((5, W1.shape[1]), jnp.float32)], axis=0)
    d1, d2, d3 = W1.shape[1], W2.shape[1], W3.shape[1]
    grid = (Ep // Eb,)
    full = lambda shape: pl.BlockSpec(shape, lambda i: (0, 0))
    return pl.pallas_call(
        _mlp2_body,
        grid=grid,
        in_specs=[
            pl.BlockSpec((Eb, F), lambda i: (i, 0)),
            pl.BlockSpec((Eb, 8), lambda i: (i, 0)),
            full((1, F)), full((1, F)), full((1, 8)), full((1, 8)),
            full((F, d1)), full((8, d1)),
            full((1, d1)), full((1, d1)), full((d1, d2)),
            full((1, d2)), full((1, d2)), full((d2, d3)),
        ],
        out_specs=pl.BlockSpec((Eb, d3), lambda i: (i, 0)),
        out_shape=jax.ShapeDtypeStruct((Ep, d3), jnp.float32),
    )(xjp, msgp, g1x, b1x, g1m, b1m, W1a, W1b,
      g2.reshape(1, -1), b2.reshape(1, -1), W2,
      g3.reshape(1, -1), b3.reshape(1, -1), W3)


def _mlp3_body(x2_ref, pos_ref, batch_ref,
               g1x, b1x, g1p, b1p, W1a, W1b, g2, b2, W2, g3, b3, W3,
               hg1, hb1, HW1, hc1, hg2, hb2, HW2, hc2, hg3, hb3, HW3, hc3,
               out_ref, pooled_ref):
    i = pl.program_id(0)
    nblk = pl.num_programs(0)

    @pl.when(i == 0)
    def _():
        pooled_ref[...] = jnp.full_like(pooled_ref, -jnp.inf)

    x2 = x2_ref[...]
    p = pos_ref[...]
    C = x2.shape[1] + 3.0
    mask3 = (lax.broadcasted_iota(jnp.int32, p.shape, 1) < 3).astype(jnp.float32)
    mu = (jnp.sum(x2, axis=1, keepdims=True) + jnp.sum(p, axis=1, keepdims=True)) / C
    var = (jnp.sum((x2 - mu) ** 2, axis=1, keepdims=True)
           + jnp.sum(((p - mu) ** 2) * mask3, axis=1, keepdims=True)) / C
    inv = lax.rsqrt(var + _EPS)
    ax = _relu((x2 - mu) * inv * g1x[...] + b1x[...])
    ap = _relu((p - mu) * inv * g1p[...] + b1p[...])
    h = (jnp.dot(ax, W1a[...], preferred_element_type=jnp.float32)
         + jnp.dot(ap, W1b[...], preferred_element_type=jnp.float32))
    h = jnp.dot(_relu(_ln(h, g2[...], b2[...])), W2[...], preferred_element_type=jnp.float32)
    g_out = jnp.dot(_relu(_ln(h, g3[...], b3[...])), W3[...], preferred_element_type=jnp.float32)

    b0 = batch_ref[...][:, :1]
    for bb in range(8):
        mask = b0 == bb
        contrib = jnp.max(jnp.where(mask, g_out, -jnp.inf), axis=0, keepdims=True)
        pooled_ref[bb:bb + 1, :] = jnp.maximum(pooled_ref[bb:bb + 1, :], contrib)

    @pl.when(i == nblk - 1)
    def _():
        pool = pooled_ref[...]
        o = jnp.dot(_relu(_ln(pool, hg1[...], hb1[...])), HW1[...],
                    preferred_element_type=jnp.float32) + hc1[...]
        o = jnp.dot(_relu(_ln(o, hg2[...], hb2[...])), HW2[...],
                    preferred_element_type=jnp.float32) + hc2[...]
        o = jnp.dot(_relu(_ln(o, hg3[...], hb3[...])), HW3[...],
                    preferred_element_type=jnp.float32) + hc3[...]
        out_ref[...] = o


def _run_mlp3(x2p, posp, batchp, pg, ph, Nb):
    Np, F2 = x2p.shape
    (g1, b1, W1), (g2, b2, W2), (g3, b3, W3) = pg
    (hg1, hb1, HW1, hc1), (hg2, hb2, HW2, hc2), (hg3, hb3, HW3, hc3) = ph
    g1x, g1p = g1[:F2].reshape(1, F2), jnp.concatenate(
        [g1[F2:], jnp.ones((5,), jnp.float32)]).reshape(1, 8)
    b1x, b1p = b1[:F2].reshape(1, F2), jnp.concatenate(
        [b1[F2:], jnp.zeros((5,), jnp.float32)]).reshape(1, 8)
    W1a = W1[:F2]
    W1b = jnp.concatenate([W1[F2:], jnp.zeros((5, W1.shape[1]), jnp.float32)], axis=0)
    d1, d2, d3 = W1.shape[1], W2.shape[1], W3.shape[1]
    e1, e2, e3 = HW1.shape[1], HW2.shape[1], HW3.shape[1]
    grid = (Np // Nb,)
    full = lambda shape: pl.BlockSpec(shape, lambda i: (0, 0))
    r2 = lambda v: v.reshape(1, -1)
    return pl.pallas_call(
        _mlp3_body,
        grid=grid,
        in_specs=[
            pl.BlockSpec((Nb, F2), lambda i: (i, 0)),
            pl.BlockSpec((Nb, 8), lambda i: (i, 0)),
            pl.BlockSpec((Nb, 8), lambda i: (i, 0)),
            full((1, F2)), full((1, F2)), full((1, 8)), full((1, 8)),
            full((F2, d1)), full((8, d1)),
            full((1, d1)), full((1, d1)), full((d1, d2)),
            full((1, d2)), full((1, d2)), full((d2, d3)),
            full((1, d3)), full((1, d3)), full((d3, e1)), full((1, e1)),
            full((1, e1)), full((1, e1)), full((e1, e2)), full((1, e2)),
            full((1, e2)), full((1, e2)), full((e2, e3)), full((1, e3)),
        ],
        out_specs=pl.BlockSpec((8, e3), lambda i: (0, 0)),
        out_shape=jax.ShapeDtypeStruct((8, e3), jnp.float32),
        scratch_shapes=[pltpu.VMEM((8, d3), jnp.float32)],
    )(x2p, posp, batchp, g1x, b1x, g1p, b1p, W1a, W1b,
      r2(g2), r2(b2), W2, r2(g3), r2(b3), W3,
      r2(hg1), r2(hb1), HW1, r2(hc1), r2(hg2), r2(hb2), HW2, r2(hc2),
      r2(hg3), r2(hb3), HW3, r2(hc3))


def _ceil_to(x, m):
    return (x + m - 1) // m * m


def kernel(pos, edge_index, batch, p_local1, p_local2, p_global, p_head):
    N = pos.shape[0]
    E = edge_index.shape[1]
    EE = E + N
    Eb = 2048
    Nb = 512
    Ep = _ceil_to(EE, Eb)
    Np = _ceil_to(N, 512)
    Np2 = Np
    loops = jnp.arange(N, dtype=edge_index.dtype)
    row = jnp.concatenate([edge_index[0], loops])
    col = jnp.concatenate([edge_index[1], loops])

    Ed = _ceil_to(E, 4096)
    rowd = jnp.full((Ed,), Np2, jnp.int32).at[:E].set(edge_index[0])
    degp = _sc_deg(rowd, Np2)

    padv = (jnp.arange(Ep - EE, dtype=jnp.int32) % N)
    rowp = jnp.concatenate([row, padv])
    colp = jnp.concatenate([col, padv])
    pospad = jnp.zeros((Np2, 3), jnp.float32).at[:N].set(pos)
    msgp = _sc_msg(rowp, colp, pospad[:, 0], pospad[:, 1], pospad[:, 2],
                   degp).reshape(Ep, 8)

    colb = jnp.concatenate([col, jnp.full((Ep - EE,), Np2, jnp.int32)])
    ids, cls_, cnt = _sc_bucket(colb, Np2)

    h = _run_mlp1(msgp, p_local1, Eb)
    x = _sc_scatter_max(h, ids, cls_, cnt, Np2, 128).reshape(Np2, 128)

    xjp = _sc_gather_rows(x, rowp)
    h2 = _run_mlp2(xjp, msgp, p_local2, Eb)
    x2 = _sc_scatter_max(h2, ids, cls_, cnt, Np2, 64).reshape(Np2, 256)

    posp = jnp.zeros((Np, 8), jnp.float32).at[:N, :3].set(pos)
    batchp = jnp.full((Np, 8), 8, jnp.int32).at[:N, :].set(batch[:, None])

    out = _run_mlp3(x2, posp, batchp, p_global, p_head, Nb)
    return out
